# baseline (speedup 1.0000x reference)
import functools

import jax
import jax.numpy as jnp
from jax import lax
from jax.experimental import pallas as pl
from jax.experimental.pallas import tpu as pltpu
from jax.experimental.pallas import tpu_sc as plsc

VOCAB = 1000000
D_MODEL = 64
BATCH = 4096
CTX = 200

NC, NS = 2, 16
NW = NC * NS
TOK = BATCH * CTX
G = 128
NROW = TOK // G
RPW = NROW // NW
K = 4
NGRP = RPW // K
NBUF = 2 * K

_mesh = plsc.VectorSubcoreMesh(core_axis_name="c", subcore_axis_name="s")


@functools.partial(
    pl.kernel,
    mesh=_mesh,
    compiler_params=pltpu.CompilerParams(use_tc_tiling_on_sc=False),
    out_type=jax.ShapeDtypeStruct((NROW, G, D_MODEL), jnp.float32),
    scratch_types=(
        [pltpu.VMEM((RPW, G), jnp.int32),
         pltpu.VMEM((NBUF, G, D_MODEL), jnp.float32)]
        + [pltpu.SemaphoreType.DMA] * NBUF
        + [pltpu.SemaphoreType.DMA] * 2
    ),
)
def _emb_sc(x2_hbm, tab_hbm, out_hbm, idx_v, rows_v, *sems):
    gs = sems[:NBUF]
    ws = sems[NBUF:]
    wid = lax.axis_index("s") * NC + lax.axis_index("c")
    row0 = wid * RPW

    pltpu.sync_copy(x2_hbm.at[pl.ds(row0, RPW)], idx_v)

    def gather(g, j, half):
        s = half * K + j
        pltpu.async_copy(
            tab_hbm.at[idx_v.at[g * K + j]], rows_v.at[s], gs[s])

    def wait_gather(g, j, half):
        s = half * K + j
        pltpu.make_async_copy(
            tab_hbm.at[idx_v.at[g * K + j]], rows_v.at[s], gs[s]).wait()

    def out_start(g, half):
        pltpu.async_copy(rows_v.at[pl.ds(half * K, K)],
                         out_hbm.at[pl.ds(row0 + g * K, K)], ws[half])

    def out_wait(g, half):
        pltpu.make_async_copy(rows_v.at[pl.ds(half * K, K)],
                              out_hbm.at[pl.ds(row0 + g * K, K)],
                              ws[half]).wait()

    for j in range(K):
        gather(0, j, 0)

    def round2(r, carry):
        for half in range(2):
            g = 2 * r + half
            other = 1 - half
            @pl.when(g + 1 < NGRP)
            def _():
                @pl.when(g >= 1)
                def _():
                    out_wait(g - 1, other)
                for j in range(K):
                    gather(g + 1, j, other)
            for j in range(K):
                wait_gather(g, j, half)
            out_start(g, half)
        return carry

    lax.fori_loop(0, NGRP // 2, round2, 0)

    out_wait(NGRP - 2, 0)
    out_wait(NGRP - 1, 1)


def kernel(x, table):
    x2 = x.reshape(NROW, G)
    out = _emb_sc(x2, table)
    return out.reshape(BATCH, CTX, D_MODEL)
